# Initial kernel scaffold; baseline (speedup 1.0000x reference)
import jax
import jax.numpy as jnp
from jax.experimental import pallas as pl


def kernel(x, W, b):
    raise NotImplementedError("write your pallas kernel here")


# speedup vs baseline: 1.1654x; 1.1654x over previous
import functools

import jax
import jax.numpy as jnp
from jax.experimental import pallas as pl


def _slice_body(inv_scale, x_ref, w_ref, b_ref, o_ref):
    xi = x_ref[0]
    xb = xi.astype(jnp.bfloat16)
    s = jax.lax.dot_general(
        xb, xb, (((1,), (1,)), ((), ())),
        preferred_element_type=jnp.float32)
    s = jnp.maximum(s * inv_scale, 0.0)
    m = jnp.max(s, axis=1, keepdims=True)
    e = jnp.exp(s - m)
    denom = jnp.sum(e, axis=1, keepdims=True)
    h = jnp.dot(e.astype(jnp.bfloat16), xb,
                preferred_element_type=jnp.float32)
    h = h / denom
    h = jnp.dot(h.astype(jnp.bfloat16), w_ref[...],
                preferred_element_type=jnp.float32)
    o_ref[0] = jnp.maximum(h + b_ref[0], 0.0)


def kernel(x, W, b):
    Bx, N, T, D = x.shape
    H = W.shape[1]
    x2 = x.reshape(Bx, N, T * D)
    Wb = W.astype(jnp.bfloat16)
    b2 = b.reshape(1, H)
    inv_scale = 1.0 / float(D) ** 0.5

    out = pl.pallas_call(
        functools.partial(_slice_body, inv_scale),
        grid=(Bx, T),
        in_specs=[
            pl.BlockSpec((1, N, D), lambda bb, t: (bb, 0, t)),
            pl.BlockSpec((D, H), lambda bb, t: (0, 0)),
            pl.BlockSpec((1, H), lambda bb, t: (0, 0)),
        ],
        out_specs=pl.BlockSpec((1, N, H), lambda bb, t: (bb, 0, t)),
        out_shape=jax.ShapeDtypeStruct((Bx, N, T * H), jnp.float32),
    )(x2, Wb, b2)
    return out.reshape(Bx, N, T, H)


# speedup vs baseline: 1.3800x; 1.1842x over previous
import functools

import jax
import jax.numpy as jnp
from jax.experimental import pallas as pl


def _batch_body(nt, inv_scale, x_ref, w_ref, b_ref, o_ref):
    w = w_ref[...]
    bias = b_ref[0]
    xall = x_ref[0]
    d = w.shape[0]
    h_dim = w.shape[1]
    for t in range(nt):
        xi = xall[:, t * d:(t + 1) * d]
        xb = xi.astype(jnp.bfloat16)
        xs = xb * jnp.bfloat16(inv_scale)
        s = jax.lax.dot_general(
            xs, xb, (((1,), (1,)), ((), ())),
            preferred_element_type=jnp.float32)
        s = jnp.maximum(s, 0.0)
        m = jnp.max(s, axis=1, keepdims=True)
        e = jnp.exp(s - m)
        denom = jnp.sum(e, axis=1, keepdims=True)
        hh = jnp.dot(e.astype(jnp.bfloat16), xb,
                     preferred_element_type=jnp.float32)
        hh = hh / denom
        hh = jnp.dot(hh.astype(jnp.bfloat16), w,
                     preferred_element_type=jnp.float32)
        o_ref[0, :, t * h_dim:(t + 1) * h_dim] = jnp.maximum(hh + bias, 0.0)


def kernel(x, W, b):
    Bx, N, T, D = x.shape
    H = W.shape[1]
    x2 = x.reshape(Bx, N, T * D)
    Wb = W.astype(jnp.bfloat16)
    b2 = b.reshape(1, H)
    inv_scale = 1.0 / float(D) ** 0.5

    out = pl.pallas_call(
        functools.partial(_batch_body, T, inv_scale),
        grid=(Bx,),
        in_specs=[
            pl.BlockSpec((1, N, T * D), lambda bb: (bb, 0, 0)),
            pl.BlockSpec((D, H), lambda bb: (0, 0)),
            pl.BlockSpec((1, H), lambda bb: (0, 0)),
        ],
        out_specs=pl.BlockSpec((1, N, T * H), lambda bb: (bb, 0, 0)),
        out_shape=jax.ShapeDtypeStruct((Bx, N, T * H), jnp.float32),
    )(x2, Wb, b2)
    return out.reshape(Bx, N, T, H)
